# baseline (speedup 1.0000x reference)
import functools

import jax
import jax.numpy as jnp
from jax import lax
from jax.experimental import pallas as pl
from jax.experimental.pallas import tpu as pltpu
from jax.experimental.pallas import tpu_sc as plsc

_B = 16384
_D = 64
_NW = 32
_CHUNK = _B // _NW
_NIDX = _CHUNK // 128
_NGROUP = _CHUNK // 16


def _body(user_h, item_h, diff_h, theta_h, a_h, b_h, w_h, ob_h, out_h,
          uidx, iidx, th, ar, br, dv, ov, colbuf, wv, obv, sem):
    cid = lax.axis_index("c")
    sid = lax.axis_index("s")
    wid = sid * 2 + cid
    base = wid * _CHUNK

    pltpu.sync_copy(user_h.at[pl.ds(wid * _NIDX, _NIDX)], uidx)
    pltpu.sync_copy(item_h.at[pl.ds(wid * _NIDX, _NIDX)], iidx)
    pltpu.sync_copy(diff_h.at[pl.ds(base, _CHUNK)], dv)
    pltpu.sync_copy(w_h, wv)
    pltpu.sync_copy(ob_h, obv)

    cps = []
    for j in range(_NIDX):
        dst = pl.ds(j * 128, 128)
        cps.append(pltpu.async_copy(theta_h.at[uidx.at[j]], th.at[dst], sem))
        cps.append(pltpu.async_copy(a_h.at[iidx.at[j]], ar.at[dst], sem))
        cps.append(pltpu.async_copy(b_h.at[iidx.at[j]], br.at[dst], sem))
    for cp in cps:
        cp.wait()

    lanes = lax.iota(jnp.int32, 16)
    tbase = lanes * 16
    zeros16 = lanes * 0
    w = wv[...]
    ob = obv[...]

    def group(g, _):
        for j in range(16):
            row = g * 16 + j
            acc = None
            for c in range(4):
                t = th[row, pl.ds(c * 16, 16)]
                av = ar[row, pl.ds(c * 16, 16)]
                sa = 1.0 / (1.0 + jnp.exp(-av))
                p = sa * t
                acc = p if acc is None else acc + p
            colbuf[pl.ds(j * 16, 16)] = acc
        dot = plsc.load_gather(colbuf, [tbase])
        for l in range(1, 16):
            dot = dot + plsc.load_gather(colbuf, [tbase + l])
        bv = plsc.load_gather(br, [g * 16 + lanes, zeros16])
        dfv = dv[pl.ds(g * 16, 16)]
        s = 1.0 / (1.0 + jnp.exp(bv - dot))
        z = s * dfv * w + ob
        ov[pl.ds(g * 16, 16)] = 1.0 / (1.0 + jnp.exp(-z))
        return 0

    lax.fori_loop(0, _NGROUP, group, 0)

    pltpu.sync_copy(ov, out_h.at[pl.ds(base, _CHUNK)])


@jax.jit
def _mirtnet_sc(user2, item2, diff, theta_w, a_w, b_w, out_W, out_b):
    mesh = plsc.VectorSubcoreMesh(core_axis_name="c", subcore_axis_name="s")
    return pl.kernel(
        _body,
        out_type=jax.ShapeDtypeStruct((_B,), jnp.float32),
        mesh=mesh,
        compiler_params=pltpu.CompilerParams(
            needs_layout_passes=False, use_tc_tiling_on_sc=False),
        scratch_types=[
            pltpu.VMEM((_NIDX, 128), jnp.int32),
            pltpu.VMEM((_NIDX, 128), jnp.int32),
            pltpu.VMEM((_CHUNK, _D), jnp.float32),
            pltpu.VMEM((_CHUNK, _D), jnp.float32),
            pltpu.VMEM((_CHUNK, 1), jnp.float32),
            pltpu.VMEM((_CHUNK,), jnp.float32),
            pltpu.VMEM((_CHUNK,), jnp.float32),
            pltpu.VMEM((256,), jnp.float32),
            pltpu.VMEM((16,), jnp.float32),
            pltpu.VMEM((16,), jnp.float32),
            pltpu.SemaphoreType.DMA,
        ],
    )(user2, item2, diff, theta_w, a_w, b_w, out_W, out_b)


def kernel(user, item, diff, theta_w, a_w, b_w, out_W, out_b):
    user2 = user.astype(jnp.int32).reshape(_NW * _NIDX, 128)
    item2 = item.astype(jnp.int32).reshape(_NW * _NIDX, 128)
    w16 = jnp.broadcast_to(out_W[0, 0], (16,))
    ob16 = jnp.broadcast_to(out_b[0], (16,))
    return _mirtnet_sc(user2, item2, diff, theta_w, a_w, b_w, w16, ob16)
